# baseline (speedup 1.0000x reference)
import functools

import jax
import jax.numpy as jnp
from jax.experimental import pallas as pl
from jax.experimental.pallas import tpu as pltpu

B, CIN, COUT, H, W, K = 2, 96, 96, 384, 384, 3
HW = H * W
S = 8192
NT = HW // S
HALO = 512


def _conv_body(w_ref, b_ref, x_ref, m_ref, o_ref, xs_ref):
    i = pl.program_id(1)
    j = jax.lax.broadcasted_iota(jnp.int32, (1, S + 2 * HALO), 1)
    raw = x_ref[0].astype(jnp.bfloat16)

    @pl.when(i != 0)
    def _():
        xs_ref[...] = raw

    @pl.when(i == 0)
    def _():
        xs_ref[...] = jnp.where(j < HALO, jnp.bfloat16(0),
                                pltpu.roll(raw, HALO, axis=1))

    @pl.when(i == NT - 1)
    def _():
        xs_ref[:, S + HALO:] = jnp.zeros((CIN, HALO), jnp.bfloat16)

    xcat = xs_ref[...]

    col = (i * S + j + (2 * W - HALO)) % W
    x_m1 = jnp.where(col == W - 1, jnp.bfloat16(0), xcat)
    x_p1 = jnp.where(col == 0, jnp.bfloat16(0), xcat)

    acc = jnp.zeros((COUT, S), jnp.float32)
    for ky in range(K):
        for kx in range(K):
            src = (x_m1, xcat, x_p1)[kx]
            d = (ky - 1) * W + (kx - 1)
            sl = jax.lax.slice(src, (0, HALO + d), (CIN, HALO + d + S))
            acc = acc + jnp.dot(w_ref[ky * K + kx], sl,
                                preferred_element_type=jnp.float32)

    m = m_ref[0].astype(jnp.float32)
    o_ref[0] = (acc + b_ref[...]) * m


@jax.jit
def kernel(x, mask, weight, bias):
    xf = x.reshape(B, CIN, HW)
    mf = mask.reshape(B * NT, 1, S)
    wt = weight.transpose(2, 3, 0, 1).reshape(K * K, COUT, CIN).astype(jnp.bfloat16)
    b2 = bias.reshape(COUT, 1)

    grid = (B, NT)
    out = pl.pallas_call(
        _conv_body,
        grid=grid,
        in_specs=[
            pl.BlockSpec((K * K, COUT, CIN), lambda b, i: (0, 0, 0)),
            pl.BlockSpec((COUT, 1), lambda b, i: (0, 0)),
            pl.BlockSpec(
                (pl.Element(1), pl.Element(CIN),
                 pl.Element(S + 2 * HALO, (0, HALO))),
                lambda b, i: (b, 0,
                              jnp.maximum(i * (S // 128) - HALO // 128, 0) * 128)),
            pl.BlockSpec((1, 1, S), lambda b, i: (b * NT + i, 0, 0)),
        ],
        out_specs=pl.BlockSpec((1, COUT, S), lambda b, i: (b, 0, i)),
        out_shape=jax.ShapeDtypeStruct((B, COUT, HW), jnp.float32),
        scratch_shapes=[pltpu.VMEM((CIN, S + 2 * HALO), jnp.bfloat16)],
        compiler_params=pltpu.CompilerParams(
            dimension_semantics=("parallel", "arbitrary")),
    )(wt, b2, xf, mf)
    return out.reshape(B, COUT, H, W)


# speedup vs baseline: 1.2078x; 1.2078x over previous
import jax
import jax.numpy as jnp
from jax.experimental import pallas as pl
from jax.experimental.pallas import tpu as pltpu

B, CIN, COUT, H, W, K = 2, 96, 96, 384, 384, 3
G = 12
TH = 16
NB = H // TH
N = TH * W


def _conv_body(w_ref, b_ref, x_ref, m_ref, o_ref, xs_ref):
    i = pl.program_id(1)

    for t in range(TH + 2):
        xs_ref[:, t * W:(t + 1) * W] = (
            x_ref[:, t].reshape(CIN, W).astype(jnp.bfloat16))

    @pl.when(i == 0)
    def _():
        for t in range(TH + 1, 0, -1):
            xs_ref[:, t * W:(t + 1) * W] = (
                x_ref[:, t - 1].reshape(CIN, W).astype(jnp.bfloat16))
        xs_ref[:, 0:W] = jnp.zeros((CIN, W), jnp.bfloat16)

    @pl.when(i == NB - 1)
    def _():
        xs_ref[:, (TH + 1) * W:] = jnp.zeros((CIN, W), jnp.bfloat16)

    col = jax.lax.broadcasted_iota(jnp.int32, (1, N), 1) % W
    acc = jnp.zeros((COUT, N), jnp.float32)
    for ky in range(K):
        a = xs_ref[:, ky * W: ky * W + N]
        for kx in range(K):
            if kx == 0:
                src = jnp.where(col == 0, jnp.bfloat16(0),
                                pltpu.roll(a, 1, axis=1))
            elif kx == 2:
                src = jnp.where(col == W - 1, jnp.bfloat16(0),
                                pltpu.roll(a, N - 1, axis=1))
            else:
                src = a
            acc = acc + jnp.dot(w_ref[ky * K + kx], src,
                                preferred_element_type=jnp.float32)

    m = m_ref[0].reshape(1, N).astype(jnp.float32)
    res = (acc + b_ref[...]) * m
    for t in range(TH):
        o_ref[:, t] = res[:, t * W:(t + 1) * W].reshape(G, 8, W)


@jax.jit
def kernel(x, mask, weight, bias):
    xg = (x.reshape(B, G, 8, H, W)
          .transpose(0, 1, 3, 2, 4)
          .reshape(B * G, H, 8, W))
    mg = mask.reshape(B, H, W)
    wt = (weight.transpose(2, 3, 0, 1)
          .reshape(K * K, COUT, CIN).astype(jnp.bfloat16))
    b2 = bias.reshape(COUT, 1)

    out = pl.pallas_call(
        _conv_body,
        grid=(B, NB),
        in_specs=[
            pl.BlockSpec((K * K, COUT, CIN), lambda b, i: (0, 0, 0)),
            pl.BlockSpec((COUT, 1), lambda b, i: (0, 0)),
            pl.BlockSpec(
                (pl.Element(G), pl.Element(TH + 2, (0, 1)),
                 pl.Element(8), pl.Element(W)),
                lambda b, i: (b * G, jnp.maximum(i * TH - 1, 0), 0, 0)),
            pl.BlockSpec((1, TH, W), lambda b, i: (b, i, 0)),
        ],
        out_specs=pl.BlockSpec((G, TH, 8, W), lambda b, i: (b, i, 0, 0)),
        out_shape=jax.ShapeDtypeStruct((B * G, H, 8, W), jnp.float32),
        scratch_shapes=[pltpu.VMEM((CIN, (TH + 2) * W), jnp.bfloat16)],
        compiler_params=pltpu.CompilerParams(
            dimension_semantics=("parallel", "arbitrary")),
    )(wt, b2, xg, mg)

    return (out.reshape(B, G, H, 8, W)
            .transpose(0, 1, 3, 2, 4)
            .reshape(B, COUT, H, W))


# speedup vs baseline: 1.9025x; 1.5752x over previous
import jax
import jax.numpy as jnp
from jax.experimental import pallas as pl
from jax.experimental.pallas import tpu as pltpu

B, CIN, COUT, H, W, K = 2, 96, 96, 384, 384, 3
TH = 24
NB = H // TH
RB = 40
OFF = 7
N = TH * W


def _conv_body(w_ref, b_ref, x_ref, m_ref, o_ref, xs_ref):
    i = pl.program_id(1)
    flat = x_ref[0].reshape(CIN, RB * W).astype(jnp.bfloat16)
    lane = jax.lax.broadcasted_iota(jnp.int32, (1, (TH + 2) * W), 1)

    @pl.when(i != 0)
    def _():
        xs_ref[...] = flat[:, OFF * W: (OFF + TH + 2) * W]

    @pl.when(i == 0)
    def _():
        xs_ref[...] = jnp.where(
            lane < W, jnp.bfloat16(0),
            pltpu.roll(flat[:, :(TH + 2) * W], W, axis=1))

    @pl.when(i == NB - 1)
    def _():
        xs_ref[:, (TH + 1) * W:] = jnp.zeros((CIN, W), jnp.bfloat16)

    col = jax.lax.broadcasted_iota(jnp.int32, (1, N), 1) % W
    acc = jnp.zeros((COUT, N), jnp.float32)
    for ky in range(K):
        a = xs_ref[:, ky * W: ky * W + N]
        for kx in range(K):
            if kx == 0:
                src = jnp.where(col == 0, jnp.bfloat16(0),
                                pltpu.roll(a, 1, axis=1))
            elif kx == 2:
                src = jnp.where(col == W - 1, jnp.bfloat16(0),
                                pltpu.roll(a, N - 1, axis=1))
            else:
                src = a
            acc = acc + jnp.dot(w_ref[ky * K + kx], src,
                                preferred_element_type=jnp.float32)

    m = m_ref[0].reshape(1, N).astype(jnp.float32)
    res = (acc + b_ref[...]) * m
    o_ref[0] = res.reshape(COUT, TH, W)


@jax.jit
def kernel(x, mask, weight, bias):
    mg = mask.reshape(B, H, W)
    wt = (weight.transpose(2, 3, 0, 1)
          .reshape(K * K, COUT, CIN).astype(jnp.bfloat16))
    b2 = bias.reshape(COUT, 1)

    return pl.pallas_call(
        _conv_body,
        grid=(B, NB),
        in_specs=[
            pl.BlockSpec((K * K, COUT, CIN), lambda b, i: (0, 0, 0)),
            pl.BlockSpec((COUT, 1), lambda b, i: (0, 0)),
            pl.BlockSpec(
                (pl.Element(1), pl.Element(CIN), pl.Element(RB, (0, 8)),
                 pl.Element(W)),
                lambda b, i: (b, 0, jnp.maximum(i * (TH // 8) - 1, 0) * 8, 0)),
            pl.BlockSpec((1, TH, W), lambda b, i: (b, i, 0)),
        ],
        out_specs=pl.BlockSpec((1, COUT, TH, W), lambda b, i: (b, 0, i, 0)),
        out_shape=jax.ShapeDtypeStruct((B, COUT, H, W), jnp.float32),
        scratch_shapes=[pltpu.VMEM((CIN, (TH + 2) * W), jnp.bfloat16)],
        compiler_params=pltpu.CompilerParams(
            dimension_semantics=("parallel", "arbitrary")),
    )(wt, b2, x, mg)


# speedup vs baseline: 1.9949x; 1.0486x over previous
import jax
import jax.numpy as jnp
from jax.experimental import pallas as pl
from jax.experimental.pallas import tpu as pltpu

B, CIN, COUT, H, W, K = 2, 96, 96, 384, 384, 3
TH = 32
NB = H // TH
RB = TH + 8
N = TH * W


def _conv_body(w_ref, b_ref, x_ref, m_ref, o_ref, xs_ref, halo_ref):
    i = pl.program_id(1)
    flat = x_ref[0].reshape(CIN, RB * W).astype(jnp.bfloat16)

    xs_ref[:, W:] = flat[:, :(TH + 1) * W]

    @pl.when(i == 0)
    def _():
        xs_ref[:, :W] = jnp.zeros((CIN, W), jnp.bfloat16)

    @pl.when(i != 0)
    def _():
        xs_ref[:, :W] = halo_ref[...]

    halo_ref[...] = flat[:, (TH - 1) * W: TH * W]

    @pl.when(i == NB - 1)
    def _():
        xs_ref[:, (TH + 1) * W:] = jnp.zeros((CIN, W), jnp.bfloat16)

    col = jax.lax.broadcasted_iota(jnp.int32, (1, N), 1) % W
    acc = jnp.zeros((COUT, N), jnp.float32)
    for ky in range(K):
        a = xs_ref[:, ky * W: ky * W + N]
        for kx in range(K):
            if kx == 0:
                src = jnp.where(col == 0, jnp.bfloat16(0),
                                pltpu.roll(a, 1, axis=1))
            elif kx == 2:
                src = jnp.where(col == W - 1, jnp.bfloat16(0),
                                pltpu.roll(a, N - 1, axis=1))
            else:
                src = a
            acc = acc + jnp.dot(w_ref[ky * K + kx], src,
                                preferred_element_type=jnp.float32)

    m = m_ref[0].reshape(1, N).astype(jnp.float32)
    res = (acc + b_ref[...]) * m
    o_ref[0] = res.reshape(COUT, TH, W)


@jax.jit
def kernel(x, mask, weight, bias):
    mg = mask.reshape(B, H, W)
    wt = (weight.transpose(2, 3, 0, 1)
          .reshape(K * K, COUT, CIN).astype(jnp.bfloat16))
    b2 = bias.reshape(COUT, 1)

    return pl.pallas_call(
        _conv_body,
        grid=(B, NB),
        in_specs=[
            pl.BlockSpec((K * K, COUT, CIN), lambda b, i: (0, 0, 0)),
            pl.BlockSpec((COUT, 1), lambda b, i: (0, 0)),
            pl.BlockSpec(
                (pl.Element(1), pl.Element(CIN), pl.Element(RB, (0, 8)),
                 pl.Element(W)),
                lambda b, i: (b, 0, i * TH, 0)),
            pl.BlockSpec((1, TH, W), lambda b, i: (b, i, 0)),
        ],
        out_specs=pl.BlockSpec((1, COUT, TH, W), lambda b, i: (b, 0, i, 0)),
        out_shape=jax.ShapeDtypeStruct((B, COUT, H, W), jnp.float32),
        scratch_shapes=[pltpu.VMEM((CIN, (TH + 2) * W), jnp.bfloat16),
                        pltpu.VMEM((CIN, W), jnp.bfloat16)],
        compiler_params=pltpu.CompilerParams(
            dimension_semantics=("arbitrary", "arbitrary")),
    )(wt, b2, x, mg)


# speedup vs baseline: 2.0605x; 1.0329x over previous
import jax
import jax.numpy as jnp
from jax.experimental import pallas as pl
from jax.experimental.pallas import tpu as pltpu

B, CIN, COUT, H, W, K = 2, 96, 96, 384, 384, 3
TH = 32
NB = H // TH
RB = TH + 8
N = TH * W


def _conv_body(w_ref, b_ref, x_ref, m_ref, o_ref, xs_ref, halo_ref):
    i = pl.program_id(1)
    flat = x_ref[0].reshape(CIN, RB * W).astype(jnp.bfloat16)

    xs_ref[:, W:] = flat[:, :(TH + 1) * W]

    @pl.when(i == 0)
    def _():
        xs_ref[:, :W] = jnp.zeros((CIN, W), jnp.bfloat16)

    @pl.when(i != 0)
    def _():
        xs_ref[:, :W] = halo_ref[...]

    halo_ref[...] = flat[:, (TH - 1) * W: TH * W]

    @pl.when(i == NB - 1)
    def _():
        xs_ref[:, (TH + 1) * W:] = jnp.zeros((CIN, W), jnp.bfloat16)

    nf = (TH + 2) * W
    colf = jax.lax.broadcasted_iota(jnp.int32, (1, nf), 1) % W
    xs = xs_ref[...]
    xsl = pltpu.roll(jnp.where(colf == W - 1, jnp.bfloat16(0), xs), 1, axis=1)
    xsr = pltpu.roll(jnp.where(colf == 0, jnp.bfloat16(0), xs), nf - 1, axis=1)

    acc = jnp.zeros((COUT, N), jnp.float32)
    for ky in range(K):
        for kx, src in ((0, xsl), (1, xs), (2, xsr)):
            acc = acc + jnp.dot(w_ref[ky * K + kx],
                                src[:, ky * W: ky * W + N],
                                preferred_element_type=jnp.float32)

    m = m_ref[0].reshape(1, N).astype(jnp.float32)
    res = (acc + b_ref[...]) * m
    o_ref[0] = res.reshape(COUT, TH, W)


@jax.jit
def kernel(x, mask, weight, bias):
    mg = mask.reshape(B, H, W)
    wt = (weight.transpose(2, 3, 0, 1)
          .reshape(K * K, COUT, CIN).astype(jnp.bfloat16))
    b2 = bias.reshape(COUT, 1)

    return pl.pallas_call(
        _conv_body,
        grid=(B, NB),
        in_specs=[
            pl.BlockSpec((K * K, COUT, CIN), lambda b, i: (0, 0, 0)),
            pl.BlockSpec((COUT, 1), lambda b, i: (0, 0)),
            pl.BlockSpec(
                (pl.Element(1), pl.Element(CIN), pl.Element(RB, (0, 8)),
                 pl.Element(W)),
                lambda b, i: (b, 0, i * TH, 0)),
            pl.BlockSpec((1, TH, W), lambda b, i: (b, i, 0)),
        ],
        out_specs=pl.BlockSpec((1, COUT, TH, W), lambda b, i: (b, 0, i, 0)),
        out_shape=jax.ShapeDtypeStruct((B, COUT, H, W), jnp.float32),
        scratch_shapes=[pltpu.VMEM((CIN, (TH + 2) * W), jnp.bfloat16),
                        pltpu.VMEM((CIN, W), jnp.bfloat16)],
        compiler_params=pltpu.CompilerParams(
            dimension_semantics=("arbitrary", "arbitrary")),
    )(wt, b2, x, mg)
